# Initial kernel scaffold; baseline (speedup 1.0000x reference)
import jax
import jax.numpy as jnp
from jax.experimental import pallas as pl


def kernel(gate_logits, attention_mask):
    raise NotImplementedError("write your pallas kernel here")


# speedup vs baseline: 10.7856x; 10.7856x over previous
import functools

import jax
import jax.numpy as jnp
from jax.experimental import pallas as pl
from jax.experimental.pallas import tpu as pltpu

_E = 64
_Q = 1.2
_EPS = 1e-5
_TOP_P = 0.75
_ENT_TH = 2.5
_DYN_COEF = 0.001
_AUX_COEF = 0.001


def _body(nb, x_ref, w_ref, out_ref, accA, accB, accSTD):
    b = pl.program_id(0)

    @pl.when(b == 0)
    def _init():
        accA[...] = jnp.zeros_like(accA)
        accB[...] = jnp.zeros_like(accB)
        accSTD[...] = jnp.zeros_like(accSTD)

    x = x_ref[...]
    w = w_ref[...]
    mx = jnp.max(x, axis=0, keepdims=True)
    e = jnp.exp(x - mx)
    z = jnp.sum(e, axis=0, keepdims=True)
    p = e / z

    pc = jnp.maximum(p, _EPS)
    pq = pc ** _Q
    sum_pq_tok = jnp.sum(pq, axis=0, keepdims=True)
    ent = (1.0 - sum_pq_tok) / (_Q - 1.0)
    high = ent >= _ENT_TH

    m1 = jnp.max(p, axis=0, keepdims=True)
    m2 = jnp.max(jnp.where(p < m1, p, -1.0), axis=0, keepdims=True)

    s = jnp.zeros_like(p)
    for j in range(_E):
        pj = p[j:j + 1, :]
        s = s + jnp.where(pj > p, pj, 0.0)

    keep = high | (p >= m2) | ((s + p) <= _TOP_P)
    rw = jnp.where(keep, p, 0.0)

    accA[...] += jnp.sum(rw * w, axis=1, keepdims=True)
    accB[...] += jnp.sum(p * w, axis=1, keepdims=True)
    accSTD[0:1, :] += jnp.sum(pc)
    accSTD[1:2, :] += jnp.sum(pq)
    accSTD[2:3, :] += jnp.sum(w)

    @pl.when(b == nb - 1)
    def _fin():
        S = accSTD[0:1, :]
        T = accSTD[1:2, :]
        D = accSTD[2:3, :]
        ent_loss = (1.0 - T / (S ** _Q)) / (_Q - 1.0)
        lb = _E * jnp.sum(accA[...] * accB[...]) / (D * D)
        out_ref[...] = _DYN_COEF * ent_loss + _AUX_COEF * lb


def kernel(gate_logits, attention_mask):
    n, e = gate_logits.shape
    bsz, seq = attention_mask.shape
    layers = n // (bsz * seq)
    tb = 2048
    while n % tb:
        tb //= 2
    nb = n // tb

    xt = gate_logits.T
    wrow = jnp.broadcast_to(
        attention_mask.reshape(-1)[None, None, :], (layers, 1, bsz * seq)
    ).reshape(1, n).astype(jnp.float32)

    out = pl.pallas_call(
        functools.partial(_body, nb),
        grid=(nb,),
        in_specs=[
            pl.BlockSpec((e, tb), lambda i: (0, i)),
            pl.BlockSpec((1, tb), lambda i: (0, i)),
        ],
        out_specs=pl.BlockSpec((1, 1), lambda i: (0, 0)),
        out_shape=jax.ShapeDtypeStruct((1, 1), jnp.float32),
        scratch_shapes=[
            pltpu.VMEM((e, 1), jnp.float32),
            pltpu.VMEM((e, 1), jnp.float32),
            pltpu.VMEM((3, 1), jnp.float32),
        ],
        compiler_params=pltpu.CompilerParams(
            dimension_semantics=("arbitrary",),
        ),
    )(xt, wrow)
    return out.reshape(())
